# Initial kernel scaffold; baseline (speedup 1.0000x reference)
import functools

import jax
import jax.numpy as jnp
from jax import lax
from jax.experimental import pallas as pl
from jax.experimental.pallas import tpu as pltpu
from jax.experimental.pallas import tpu_sc as plsc

N = 50000
OUT = 64
E = 400000
HALF = N // 2

CHUNK = 2048
NCHUNKS = 400
E_PAD = NCHUNKS * CHUNK
N_BATCH = CHUNK // 128
CH_PER_TILE = NCHUNKS // 16

WB_CHUNKS = (HALF + 127) // 128
ZSPAN = 1568


def _sc_body(rows_h, cols_h, vals_h, w_h, out_h,
             acc, erow, ecol, evalb, gbuf, zbuf):
    c = lax.axis_index("c")
    s = lax.axis_index("s")
    lo = c * HALF

    zero16 = jnp.zeros((16,), jnp.float32)

    def zb(i, carry):
        r = i // 4
        q = (i % 4) * 16
        zbuf[r, pl.ds(q, 16)] = zero16
        return carry

    lax.fori_loop(0, 512, zb, 0)

    def zc(k, carry):
        base = jnp.minimum(s * ZSPAN + k * 128, HALF - 128)
        pltpu.sync_copy(zbuf, acc.at[pl.ds(base, 128)])
        return carry

    lax.fori_loop(0, 13, zc, 0)
    plsc.subcore_barrier()

    def chunk_body(ch, carry):
        cg = s * CH_PER_TILE + ch
        pltpu.sync_copy(rows_h.at[cg], erow)
        pltpu.sync_copy(cols_h.at[cg], ecol)
        pltpu.sync_copy(vals_h.at[cg], evalb)

        def prep(p, cy):
            jj = p // 8
            kk = (p % 8) * 16
            r16 = erow[jj, pl.ds(kk, 16)]
            local = r16 - lo
            neg = (local < 0).astype(jnp.int32)
            big = (local >= HALF).astype(jnp.int32)
            own = (neg + big) == 0
            wrap = local + HALF * neg - HALF * big
            v16 = evalb[jj, pl.ds(kk, 16)]
            erow[jj, pl.ds(kk, 16)] = wrap
            evalb[jj, pl.ds(kk, 16)] = jnp.where(own, v16, 0.0)
            return cy

        lax.fori_loop(0, 128, prep, 0)

        def batch(jb, cy):
            pltpu.sync_copy(w_h.at[ecol.at[jb]], gbuf)

            def scale(rr, cz):
                v = plsc.load_gather(
                    evalb,
                    [jnp.full((16,), jb, jnp.int32),
                     jnp.full((16,), rr, jnp.int32)])
                for q in range(4):
                    gbuf[rr, pl.ds(q * 16, 16)] = gbuf[rr, pl.ds(q * 16, 16)] * v
                return cz

            lax.fori_loop(0, 128, scale, 0)
            pltpu.sync_copy(gbuf, acc.at[erow.at[jb]], add=True)
            return cy

        lax.fori_loop(0, N_BATCH, batch, 0)
        return carry

    lax.fori_loop(0, CH_PER_TILE, chunk_body, 0)
    plsc.subcore_barrier()

    def wb(k, carry):
        chn = s + k * 16
        base = jnp.minimum(chn * 128, HALF - 128)
        pltpu.sync_copy(acc.at[pl.ds(base, 128)], zbuf)

        def relu(i, cy):
            r = i // 4
            q = (i % 4) * 16
            zbuf[r, pl.ds(q, 16)] = jnp.maximum(zbuf[r, pl.ds(q, 16)], 0.0)
            return cy

        lax.fori_loop(0, 512, relu, 0)
        pltpu.sync_copy(zbuf, out_h.at[pl.ds(lo + base, 128)])
        return carry

    nk = jnp.where(s < WB_CHUNKS - 12 * 16, 13, 12)
    lax.fori_loop(0, nk, wb, 0)


@jax.jit
def kernel(features, edge_index0, edge_values0, edge_index1, edge_values1, W):
    del features
    pad = E_PAD - 2 * E
    pad_rows = (jnp.arange(pad, dtype=jnp.int32) % N)
    pad_cols = (jnp.arange(pad, dtype=jnp.int32) % (2 * N))
    rows = jnp.concatenate([
        edge_index0[0].astype(jnp.int32),
        edge_index1[0].astype(jnp.int32), pad_rows])
    cols = jnp.concatenate([
        edge_index0[1].astype(jnp.int32),
        edge_index1[1].astype(jnp.int32) + N, pad_cols])
    vals = jnp.concatenate([
        edge_values0, edge_values1, jnp.zeros((pad,), jnp.float32)])
    rows3 = rows.reshape(NCHUNKS, 16, 128)
    cols3 = cols.reshape(NCHUNKS, 16, 128)
    vals3 = vals.reshape(NCHUNKS, 16, 128)

    mesh = plsc.VectorSubcoreMesh(core_axis_name="c", subcore_axis_name="s")
    f = pl.kernel(
        _sc_body,
        out_type=jax.ShapeDtypeStruct((N, OUT), jnp.float32),
        mesh=mesh,
        scratch_types=[
            pltpu.VMEM_SHARED((HALF, OUT), jnp.float32),
            pltpu.VMEM((16, 128), jnp.int32),
            pltpu.VMEM((16, 128), jnp.int32),
            pltpu.VMEM((16, 128), jnp.float32),
            pltpu.VMEM((128, OUT), jnp.float32),
            pltpu.VMEM((128, OUT), jnp.float32),
        ],
    )
    return f(rows3, cols3, vals3, W)


# speedup vs baseline: 3.3104x; 3.3104x over previous
import functools

import jax
import jax.numpy as jnp
from jax import lax
from jax.experimental import pallas as pl
from jax.experimental.pallas import tpu as pltpu
from jax.experimental.pallas import tpu_sc as plsc

N = 50000
OUT = 64
E = 400000
HALF = N // 2

CHUNK = 2048
NCHUNKS = 400
E_PAD = NCHUNKS * CHUNK
N_BATCH = CHUNK // 128
CH_PER_TILE = NCHUNKS // 16

WB_CHUNKS = (HALF + 127) // 128
ZSPAN = 1568


def _sc_body(rows_h, cols_h, vals_h, w_h, out_h,
             acc, erow, ecol, evalb, gbuf, zbuf):
    c = lax.axis_index("c")
    s = lax.axis_index("s")
    lo = c * HALF

    zero16 = jnp.zeros((16,), jnp.float32)

    def zb(i, carry):
        r = i // 4
        q = (i % 4) * 16
        zbuf[r, pl.ds(q, 16)] = zero16
        return carry

    lax.fori_loop(0, 512, zb, 0)

    def zc(k, carry):
        base = jnp.minimum(s * ZSPAN + k * 128, HALF - 128)
        pltpu.sync_copy(zbuf, acc.at[pl.ds(base, 128)])
        return carry

    lax.fori_loop(0, 13, zc, 0)
    plsc.subcore_barrier()

    def chunk_body(ch, carry):
        cg = s * CH_PER_TILE + ch
        pltpu.sync_copy(rows_h.at[cg], erow)
        pltpu.sync_copy(cols_h.at[cg], ecol)
        pltpu.sync_copy(vals_h.at[cg], evalb)

        zero16i = jnp.zeros((16,), jnp.int32)
        half16 = jnp.full((16,), HALF, jnp.int32)
        zero16f = jnp.zeros((16,), jnp.float32)
        lo16 = jnp.full((16,), lo, jnp.int32)

        def prep(p, cy):
            jj = p // 8
            kk = (p % 8) * 16
            r16 = erow[jj, pl.ds(kk, 16)]
            local = r16 - lo16
            neg = local < zero16i
            big = local >= half16
            own = jnp.logical_not(jnp.logical_or(neg, big))
            wrap = (local + jnp.where(neg, half16, zero16i)
                    - jnp.where(big, half16, zero16i))
            v16 = evalb[jj, pl.ds(kk, 16)]
            erow[jj, pl.ds(kk, 16)] = wrap
            evalb[jj, pl.ds(kk, 16)] = jnp.where(own, v16, zero16f)
            return cy

        lax.fori_loop(0, 128, prep, 0)

        def batch(jb, cy):
            pltpu.sync_copy(w_h.at[ecol.at[jb]], gbuf)

            def scale(rr, cz):
                v = plsc.load_gather(
                    evalb,
                    [jnp.full((16,), jb, jnp.int32),
                     jnp.full((16,), rr, jnp.int32)])
                for q in range(4):
                    gbuf[rr, pl.ds(q * 16, 16)] = gbuf[rr, pl.ds(q * 16, 16)] * v
                return cz

            lax.fori_loop(0, 128, scale, 0)
            pltpu.sync_copy(gbuf, acc.at[erow.at[jb]], add=True)
            return cy

        lax.fori_loop(0, N_BATCH, batch, 0)
        return carry

    lax.fori_loop(0, CH_PER_TILE, chunk_body, 0)
    plsc.subcore_barrier()

    def wb(k, carry):
        chn = s + k * 16
        base = jnp.minimum(chn * 128, HALF - 128)
        pltpu.sync_copy(acc.at[pl.ds(base, 128)], zbuf)

        zero16f = jnp.zeros((16,), jnp.float32)

        def relu(i, cy):
            r = i // 4
            q = (i % 4) * 16
            zbuf[r, pl.ds(q, 16)] = jnp.maximum(zbuf[r, pl.ds(q, 16)], zero16f)
            return cy

        lax.fori_loop(0, 512, relu, 0)
        pltpu.sync_copy(zbuf, out_h.at[pl.ds(lo + base, 128)])
        return carry

    nk = jnp.where(s < WB_CHUNKS - 12 * 16, 13, 12)
    lax.fori_loop(0, nk, wb, 0)


@jax.jit
def kernel(features, edge_index0, edge_values0, edge_index1, edge_values1, W):
    del features
    pad = E_PAD - 2 * E
    pad_rows = (jnp.arange(pad, dtype=jnp.int32) % N)
    pad_cols = (jnp.arange(pad, dtype=jnp.int32) % (2 * N))
    rows = jnp.concatenate([
        edge_index0[0].astype(jnp.int32),
        edge_index1[0].astype(jnp.int32), pad_rows])
    cols = jnp.concatenate([
        edge_index0[1].astype(jnp.int32),
        edge_index1[1].astype(jnp.int32) + N, pad_cols])
    vals = jnp.concatenate([
        edge_values0, edge_values1, jnp.zeros((pad,), jnp.float32)])
    rows3 = rows.reshape(NCHUNKS, 16, 128)
    cols3 = cols.reshape(NCHUNKS, 16, 128)
    vals3 = vals.reshape(NCHUNKS, 16, 128)

    mesh = plsc.VectorSubcoreMesh(core_axis_name="c", subcore_axis_name="s")
    f = pl.kernel(
        _sc_body,
        out_type=jax.ShapeDtypeStruct((N, OUT), jnp.float32),
        mesh=mesh,
        compiler_params=pltpu.CompilerParams(
            needs_layout_passes=False, use_tc_tiling_on_sc=False),
        scratch_types=[
            pltpu.VMEM_SHARED((HALF, OUT), jnp.float32),
            pltpu.VMEM((16, 128), jnp.int32),
            pltpu.VMEM((16, 128), jnp.int32),
            pltpu.VMEM((16, 128), jnp.float32),
            pltpu.VMEM((128, OUT), jnp.float32),
            pltpu.VMEM((128, OUT), jnp.float32),
        ],
    )
    return f(rows3, cols3, vals3, W)


# speedup vs baseline: 6.3483x; 1.9177x over previous
import functools

import jax
import jax.numpy as jnp
from jax import lax
from jax.experimental import pallas as pl
from jax.experimental.pallas import tpu as pltpu
from jax.experimental.pallas import tpu_sc as plsc

N = 50000
OUT = 64
E = 400000
HALF = N // 2

CHUNK = 2048
NCHUNKS = 400
E_PAD = NCHUNKS * CHUNK
N_BATCH = CHUNK // 128
CH_PER_TILE = NCHUNKS // 16

WB_CHUNKS = (HALF + 127) // 128
ZSPAN = 1568


def _sc_body(rows_h, cols_h, vals_h, w_h, out_h,
             acc, erow, ecol, evalb, gbuf0, gbuf1, zbuf,
             gsem0, gsem1, ssem0, ssem1):
    c = lax.axis_index("c")
    s = lax.axis_index("s")
    lo = c * HALF

    zero16 = jnp.zeros((16,), jnp.float32)

    def zb(i, carry):
        r = i // 4
        q = (i % 4) * 16
        zbuf[r, pl.ds(q, 16)] = zero16
        return carry

    lax.fori_loop(0, 512, zb, 0)

    def zc(k, carry):
        base = jnp.minimum(s * ZSPAN + k * 128, HALF - 128)
        pltpu.sync_copy(zbuf, acc.at[pl.ds(base, 128)])
        return carry

    lax.fori_loop(0, 13, zc, 0)
    plsc.subcore_barrier()

    def chunk_body(ch, carry):
        cg = s * CH_PER_TILE + ch
        pltpu.sync_copy(rows_h.at[cg], erow)
        pltpu.sync_copy(cols_h.at[cg], ecol)
        pltpu.sync_copy(vals_h.at[cg], evalb)

        zero16i = jnp.zeros((16,), jnp.int32)
        half16 = jnp.full((16,), HALF, jnp.int32)
        zero16f = jnp.zeros((16,), jnp.float32)
        lo16 = jnp.full((16,), lo, jnp.int32)

        @plsc.parallel_loop(0, 128, unroll=4)
        def prep(p):
            jj = p // 8
            kk = (p % 8) * 16
            r16 = erow[jj, pl.ds(kk, 16)]
            local = r16 - lo16
            neg = local < zero16i
            big = local >= half16
            own = jnp.logical_not(jnp.logical_or(neg, big))
            wrap = (local + jnp.where(neg, half16, zero16i)
                    - jnp.where(big, half16, zero16i))
            v16 = evalb[jj, pl.ds(kk, 16)]
            erow[jj, pl.ds(kk, 16)] = wrap
            evalb[jj, pl.ds(kk, 16)] = jnp.where(own, v16, zero16f)

        bufs = (gbuf0, gbuf1)
        gsems = (gsem0, gsem1)
        ssems = (ssem0, ssem1)
        gh = [None] * N_BATCH
        sh = [None] * N_BATCH
        gh[0] = pltpu.async_copy(w_h.at[ecol.at[0]], bufs[0], gsems[0])
        for jb in range(N_BATCH):
            p = jb % 2
            q = 1 - p
            if jb + 1 < N_BATCH:
                if jb >= 1:
                    sh[jb - 1].wait()
                gh[jb + 1] = pltpu.async_copy(
                    w_h.at[ecol.at[jb + 1]], bufs[q], gsems[q])
            gh[jb].wait()
            buf = bufs[p]

            @plsc.parallel_loop(0, 128, unroll=4)
            def scale(rr):
                v = plsc.load_gather(
                    evalb,
                    [jnp.full((16,), jb, jnp.int32),
                     jnp.full((16,), rr, jnp.int32)])
                for w in range(4):
                    buf[rr, pl.ds(w * 16, 16)] = buf[rr, pl.ds(w * 16, 16)] * v

            sh[jb] = pltpu.async_copy(
                buf, acc.at[erow.at[jb]], ssems[p], add=True)
        sh[N_BATCH - 2].wait()
        sh[N_BATCH - 1].wait()
        return carry

    lax.fori_loop(0, CH_PER_TILE, chunk_body, 0)
    plsc.subcore_barrier()

    def wb(k, carry):
        chn = s + k * 16
        base = jnp.minimum(chn * 128, HALF - 128)
        pltpu.sync_copy(acc.at[pl.ds(base, 128)], zbuf)

        zero16f = jnp.zeros((16,), jnp.float32)

        def relu(i, cy):
            r = i // 4
            q = (i % 4) * 16
            zbuf[r, pl.ds(q, 16)] = jnp.maximum(zbuf[r, pl.ds(q, 16)], zero16f)
            return cy

        lax.fori_loop(0, 512, relu, 0)
        pltpu.sync_copy(zbuf, out_h.at[pl.ds(lo + base, 128)])
        return carry

    nk = jnp.where(s < WB_CHUNKS - 12 * 16, 13, 12)
    lax.fori_loop(0, nk, wb, 0)


@jax.jit
def kernel(features, edge_index0, edge_values0, edge_index1, edge_values1, W):
    del features
    pad = E_PAD - 2 * E
    pad_rows = (jnp.arange(pad, dtype=jnp.int32) % N)
    pad_cols = (jnp.arange(pad, dtype=jnp.int32) % (2 * N))
    rows = jnp.concatenate([
        edge_index0[0].astype(jnp.int32),
        edge_index1[0].astype(jnp.int32), pad_rows])
    cols = jnp.concatenate([
        edge_index0[1].astype(jnp.int32),
        edge_index1[1].astype(jnp.int32) + N, pad_cols])
    vals = jnp.concatenate([
        edge_values0, edge_values1, jnp.zeros((pad,), jnp.float32)])
    rows3 = rows.reshape(NCHUNKS, 16, 128)
    cols3 = cols.reshape(NCHUNKS, 16, 128)
    vals3 = vals.reshape(NCHUNKS, 16, 128)

    mesh = plsc.VectorSubcoreMesh(core_axis_name="c", subcore_axis_name="s")
    f = pl.kernel(
        _sc_body,
        out_type=jax.ShapeDtypeStruct((N, OUT), jnp.float32),
        mesh=mesh,
        compiler_params=pltpu.CompilerParams(
            needs_layout_passes=False, use_tc_tiling_on_sc=False),
        scratch_types=[
            pltpu.VMEM_SHARED((HALF, OUT), jnp.float32),
            pltpu.VMEM((16, 128), jnp.int32),
            pltpu.VMEM((16, 128), jnp.int32),
            pltpu.VMEM((16, 128), jnp.float32),
            pltpu.VMEM((128, OUT), jnp.float32),
            pltpu.VMEM((128, OUT), jnp.float32),
            pltpu.VMEM((128, OUT), jnp.float32),
            pltpu.SemaphoreType.DMA,
            pltpu.SemaphoreType.DMA,
            pltpu.SemaphoreType.DMA,
            pltpu.SemaphoreType.DMA,
        ],
    )
    return f(rows3, cols3, vals3, W)


# speedup vs baseline: 6.5053x; 1.0247x over previous
import jax
import jax.numpy as jnp
from jax import lax
from jax.experimental import pallas as pl
from jax.experimental.pallas import tpu as pltpu
from jax.experimental.pallas import tpu_sc as plsc

N = 50000
OUT = 64
E = 400000
COL = OUT // 2

CHUNK = 2048
NCHUNKS = 400
E_PAD = NCHUNKS * CHUNK
N_BATCH = CHUNK // 128
CH_PER_TILE = NCHUNKS // 16

WB_CHUNKS = (N + 127) // 128
ZSPAN = 3136


def _sc_body(rows_h, cols_h, vals_h, w2_h, out2_h,
             acc, erow, ecol, evalb, gbuf0, gbuf1, zbuf,
             gsem0, gsem1, ssem0, ssem1):
    c = lax.axis_index("c")
    s = lax.axis_index("s")
    wv = w2_h.at[c]
    ov = out2_h.at[c]

    zero16 = jnp.zeros((16,), jnp.float32)

    @plsc.parallel_loop(0, 128 * COL // 16, unroll=4)
    def zb(i):
        r = i // (COL // 16)
        q = (i % (COL // 16)) * 16
        zbuf[r, pl.ds(q, 16)] = zero16

    def zc(k, carry):
        base = jnp.minimum(s * ZSPAN + k * 128, N - 128)
        pltpu.sync_copy(zbuf, acc.at[pl.ds(base, 128)])
        return carry

    lax.fori_loop(0, 25, zc, 0)
    plsc.subcore_barrier()

    def chunk_body(ch, carry):
        cg = s * CH_PER_TILE + ch
        pltpu.sync_copy(rows_h.at[cg], erow)
        pltpu.sync_copy(cols_h.at[cg], ecol)
        pltpu.sync_copy(vals_h.at[cg], evalb)

        bufs = (gbuf0, gbuf1)
        gsems = (gsem0, gsem1)
        ssems = (ssem0, ssem1)
        gh = [None] * N_BATCH
        sh = [None] * N_BATCH
        gh[0] = pltpu.async_copy(wv.at[ecol.at[0]], bufs[0], gsems[0])
        for jb in range(N_BATCH):
            p = jb % 2
            q = 1 - p
            if jb + 1 < N_BATCH:
                if jb >= 1:
                    sh[jb - 1].wait()
                gh[jb + 1] = pltpu.async_copy(
                    wv.at[ecol.at[jb + 1]], bufs[q], gsems[q])
            gh[jb].wait()
            buf = bufs[p]

            @plsc.parallel_loop(0, 128, unroll=4)
            def scale(rr):
                v = plsc.load_gather(
                    evalb,
                    [jnp.full((16,), jb, jnp.int32),
                     jnp.full((16,), rr, jnp.int32)])
                for w in range(COL // 16):
                    buf[rr, pl.ds(w * 16, 16)] = buf[rr, pl.ds(w * 16, 16)] * v

            sh[jb] = pltpu.async_copy(
                buf, acc.at[erow.at[jb]], ssems[p], add=True)
        sh[N_BATCH - 2].wait()
        sh[N_BATCH - 1].wait()
        return carry

    lax.fori_loop(0, CH_PER_TILE, chunk_body, 0)
    plsc.subcore_barrier()

    zero16f = jnp.zeros((16,), jnp.float32)

    def wb(k, carry):
        chn = s + k * 16
        base = jnp.minimum(chn * 128, N - 128)
        pltpu.sync_copy(acc.at[pl.ds(base, 128)], zbuf)

        @plsc.parallel_loop(0, 128 * COL // 16, unroll=4)
        def relu(i):
            r = i // (COL // 16)
            q = (i % (COL // 16)) * 16
            zbuf[r, pl.ds(q, 16)] = jnp.maximum(zbuf[r, pl.ds(q, 16)], zero16f)

        pltpu.sync_copy(zbuf, ov.at[pl.ds(base, 128)])
        return carry

    nk = jnp.where(s < WB_CHUNKS - 24 * 16, 25, 24)
    lax.fori_loop(0, nk, wb, 0)


@jax.jit
def kernel(features, edge_index0, edge_values0, edge_index1, edge_values1, W):
    del features
    pad = E_PAD - 2 * E
    pad_rows = (jnp.arange(pad, dtype=jnp.int32) % N)
    pad_cols = (jnp.arange(pad, dtype=jnp.int32) % (2 * N))
    rows = jnp.concatenate([
        edge_index0[0].astype(jnp.int32),
        edge_index1[0].astype(jnp.int32), pad_rows])
    cols = jnp.concatenate([
        edge_index0[1].astype(jnp.int32),
        edge_index1[1].astype(jnp.int32) + N, pad_cols])
    vals = jnp.concatenate([
        edge_values0, edge_values1, jnp.zeros((pad,), jnp.float32)])
    rows3 = rows.reshape(NCHUNKS, 16, 128)
    cols3 = cols.reshape(NCHUNKS, 16, 128)
    vals3 = vals.reshape(NCHUNKS, 16, 128)
    w2 = jnp.stack([W[:, :COL], W[:, COL:]])

    mesh = plsc.VectorSubcoreMesh(core_axis_name="c", subcore_axis_name="s")
    f = pl.kernel(
        _sc_body,
        out_type=jax.ShapeDtypeStruct((2, N, COL), jnp.float32),
        mesh=mesh,
        compiler_params=pltpu.CompilerParams(
            needs_layout_passes=False, use_tc_tiling_on_sc=False),
        scratch_types=[
            pltpu.VMEM_SHARED((N, COL), jnp.float32),
            pltpu.VMEM((16, 128), jnp.int32),
            pltpu.VMEM((16, 128), jnp.int32),
            pltpu.VMEM((16, 128), jnp.float32),
            pltpu.VMEM((128, COL), jnp.float32),
            pltpu.VMEM((128, COL), jnp.float32),
            pltpu.VMEM((128, COL), jnp.float32),
            pltpu.SemaphoreType.DMA,
            pltpu.SemaphoreType.DMA,
            pltpu.SemaphoreType.DMA,
            pltpu.SemaphoreType.DMA,
        ],
    )
    o2 = f(rows3, cols3, vals3, w2)
    return jnp.concatenate([o2[0], o2[1]], axis=1)
